# baseline (speedup 1.0000x reference)
import functools

import jax
import jax.numpy as jnp
from jax import lax
from jax.experimental import pallas as pl
from jax.experimental.pallas import tpu as pltpu
from jax.experimental.pallas import tpu_sc as plsc

N = 4096
D = 128
N_RBF = 64
NUM_SPECIES = 8
CELL = 1.0
CUTOFF = 0.12
CUT2 = CUTOFF * CUTOFF
GAMMA = 0.5 / ((CUTOFF / N_RBF) ** 2)

NC = 2
NS = 16
L = 16
NW = NC * NS
ROWS_PER_W = N // NW
ECAP_W = 4096
E_CAP = NW * ECAP_W
HALF = N // 2
RING_CHUNKS = HALF // L
SC_SLOTS = NS * ECAP_W
TRASH_SP = SC_SLOTS

CB = 128
RPS = N // NS

_mesh = plsc.VectorSubcoreMesh(core_axis_name="c", subcore_axis_name="s")


def _pair_body(qx_h, qy_h, qz_h, sx_h, sy_h, sz_h,
               ij_h, dq_h, cnt_h,
               qx_v, qy_v, qz_v, sx_v, sy_v, sz_v,
               stage, zb_v, zf_v, vv_ij, vv_d, cnt_v,
               sh_ij, sh_dq, sem_i, sem_d):
    cid = lax.axis_index("c").astype(jnp.int32)
    sid = lax.axis_index("s").astype(jnp.int32)
    wid = sid * NC + cid
    row0 = wid * ROWS_PER_W
    s0 = sid * ECAP_W

    pltpu.sync_copy(qx_h, qx_v.at[pl.ds(0, N)])
    pltpu.sync_copy(qy_h, qy_v.at[pl.ds(0, N)])
    pltpu.sync_copy(qz_h, qz_v.at[pl.ds(0, N)])
    pltpu.sync_copy(qx_h.at[pl.ds(0, HALF + L)], qx_v.at[pl.ds(N, HALF + L)])
    pltpu.sync_copy(qy_h.at[pl.ds(0, HALF + L)], qy_v.at[pl.ds(N, HALF + L)])
    pltpu.sync_copy(qz_h.at[pl.ds(0, HALF + L)], qz_v.at[pl.ds(N, HALF + L)])
    pltpu.sync_copy(sx_h.at[pl.ds(row0, ROWS_PER_W)], sx_v)
    pltpu.sync_copy(sy_h.at[pl.ds(row0, ROWS_PER_W)], sy_v)
    pltpu.sync_copy(sz_h.at[pl.ds(row0, ROWS_PER_W)], sz_v)

    stage[...] = jnp.zeros((160,), jnp.int32)
    zb_v[...] = jnp.zeros((512,), jnp.int32)
    zf_v[...] = jnp.zeros((512,), jnp.float32)

    def zfill(t, _):
        off = s0 + t * 512
        pltpu.sync_copy(zb_v, sh_ij.at[pl.ds(off, 512)])
        pltpu.sync_copy(zf_v, sh_dq.at[pl.ds(off, 512)])
        return _

    lax.fori_loop(0, ECAP_W // 512, zfill, 0)
    iota = lax.iota(jnp.int32, L)

    def ibody(k, cnt):
        i = row0 + k
        qix = sx_v[k, pl.ds(0, L)]
        qiy = sy_v[k, pl.ds(0, L)]
        qiz = sz_v[k, pl.ds(0, L)]
        ihi = jnp.full((L,), i * 4096, jnp.int32)
        anti_keep = i < HALF

        def dist(o):
            dx = qx_v[pl.ds(o, L)] - qix
            dy = qy_v[pl.ds(o, L)] - qiy
            dz = qz_v[pl.ds(o, L)] - qiz
            ax = 1.0 - jnp.abs(dx)
            ay = 1.0 - jnp.abs(dy)
            az = 1.0 - jnp.abs(dz)
            w = jnp.minimum(dx * dx, ax * ax)
            w = w + jnp.minimum(dy * dy, ay * ay)
            w = w + jnp.minimum(dz * dz, az * az)
            return w

        def emit(o, dsq, m, c_):
            mi = jnp.where(m, 1, 0)
            s = mi
            p = mi
            for sh in (1, 2, 4, 8):
                stage[pl.ds(80, L)] = s
                s = s + stage[pl.ds(80 + sh, L)]
                stage[pl.ds(48, L)] = p
                p = p + stage[pl.ds(48 - sh, L)]
            total = s[0]
            room = jnp.minimum(total, ECAP_W - c_)

            @pl.when(total > 0)
            def _():
                ok = m & (p <= jnp.full((L,), ECAP_W - c_, jnp.int32))
                idx = jnp.where(ok, jnp.full((L,), s0 + c_ - 1, jnp.int32) + p,
                                jnp.full((L,), TRASH_SP, jnp.int32))
                jv = (o + iota) & (N - 1)
                vv_ij[...] = ihi | jv
                vv_d[...] = dsq
                pltpu.async_copy(vv_ij, sh_ij.at[idx], sem_i, add=True).wait()
                pltpu.async_copy(vv_d, sh_dq.at[idx], sem_d, add=True).wait()

            return c_ + room

        def upair(u, c):
            oA = i + 1 + u * 2 * L
            oB = oA + L
            dsqA = dist(oA)
            dsqB = dist(oB)
            mA = (dsqA < CUT2) & (dsqA != 0.0)
            mB = (dsqB < CUT2) & (dsqB != 0.0)
            lim = jnp.where((u < RING_CHUNKS // 2 - 1) | anti_keep, 16, 15)
            mB = mB & (iota < jnp.full((L,), lim, jnp.int32))
            mor = jnp.where(mA | mB, 1, 0)
            s = mor
            for sh in (1, 2, 4, 8):
                stage[pl.ds(112, L)] = s
                s = s | stage[pl.ds(112 + sh, L)]
            anyhit = s[0]

            def scat2(c_):
                c2 = emit(oA, dsqA, mA, c_)
                return emit(oB, dsqB, mB, c2)

            return lax.cond(anyhit > 0, scat2, lambda c_: c_, c)

        return lax.fori_loop(0, RING_CHUNKS // 2, upair, cnt)

    cnt = lax.fori_loop(0, ROWS_PER_W, ibody, jnp.int32(0))
    cnt_v[...] = jnp.full((L,), cnt, jnp.int32)
    pltpu.sync_copy(cnt_v, cnt_h.at[wid])
    e0 = wid * ECAP_W
    pltpu.sync_copy(sh_ij.at[pl.ds(s0, ECAP_W)], ij_h.at[pl.ds(e0, ECAP_W)])
    pltpu.sync_copy(sh_dq.at[pl.ds(s0, ECAP_W)], dq_h.at[pl.ds(e0, ECAP_W)])


_pair_search = pl.kernel(
    _pair_body,
    out_type=[
        jax.ShapeDtypeStruct((E_CAP,), jnp.int32),
        jax.ShapeDtypeStruct((E_CAP,), jnp.float32),
        jax.ShapeDtypeStruct((NW, L), jnp.int32),
    ],
    mesh=_mesh,
    scratch_types=[
        pltpu.VMEM((N + HALF + L,), jnp.float32),
        pltpu.VMEM((N + HALF + L,), jnp.float32),
        pltpu.VMEM((N + HALF + L,), jnp.float32),
        pltpu.VMEM((ROWS_PER_W, L), jnp.float32),
        pltpu.VMEM((ROWS_PER_W, L), jnp.float32),
        pltpu.VMEM((ROWS_PER_W, L), jnp.float32),
        pltpu.VMEM((160,), jnp.int32),
        pltpu.VMEM((512,), jnp.int32),
        pltpu.VMEM((512,), jnp.float32),
        pltpu.VMEM((L,), jnp.int32),
        pltpu.VMEM((L,), jnp.float32),
        pltpu.VMEM((L,), jnp.int32),
        pltpu.VMEM_SHARED((SC_SLOTS + L,), jnp.int32),
        pltpu.VMEM_SHARED((SC_SLOTS + L,), jnp.float32),
        pltpu.SemaphoreType.DMA,
        pltpu.SemaphoreType.DMA,
    ],
)


def _msg_body(h_h, ij_h, filt_h, cnt_h, p2_h,
              ijv, iviA, ivjA, iviB, ivjB, hjA, hiA, hjB, hiB, ft, cntv, shared,
              semA1, semA2, semB1, semB2):
    cid = lax.axis_index("c").astype(jnp.int32)
    sid = lax.axis_index("s").astype(jnp.int32)
    wid = sid * NC + cid
    e0 = wid * ECAP_W

    hjA[...] = jnp.zeros((CB, D), jnp.float32)
    pltpu.sync_copy(hjA, shared.at[pl.ds(sid * RPS, CB)])
    pltpu.sync_copy(hjA, shared.at[pl.ds(sid * RPS + CB, CB)])
    plsc.subcore_barrier()

    pltpu.sync_copy(cnt_h.at[wid], cntv)
    c = jnp.minimum(cntv[pl.ds(0, L)][0], ECAP_W)
    nch = (c + CB - 1) // CB
    npairs = jnp.maximum(1, (nch + 1) // 2)

    def load_unpack(k, ivi, ivj):
        pltpu.sync_copy(ij_h.at[pl.ds(e0 + k * CB, CB)], ijv)
        for g in range(CB // L):
            v = ijv[pl.ds(g * L, L)]
            ivi[pl.ds(g * L, L)] = lax.shift_right_logical(v, 12)
            ivj[pl.ds(g * L, L)] = v & (N - 1)

    def mul_scatter(k, hj, hi, ivi, ivj):
        pltpu.sync_copy(filt_h.at[pl.ds(e0 + k * CB, CB)], ft)

        def rowbody(r, __):
            for col in range(0, D, L):
                hj[r, pl.ds(col, L)] = hj[r, pl.ds(col, L)] * ft[r, pl.ds(col, L)]
                hi[r, pl.ds(col, L)] = hi[r, pl.ds(col, L)] * ft[r, pl.ds(col, L)]
            return __

        lax.fori_loop(0, CB, rowbody, 0)
        pltpu.sync_copy(hj, shared.at[ivi], add=True)
        pltpu.sync_copy(hi, shared.at[ivj], add=True)

    load_unpack(jnp.int32(0), iviA, ivjA)
    pltpu.async_copy(h_h.at[ivjA], hjA, semA1)
    pltpu.async_copy(h_h.at[iviA], hiA, semA2)

    def pairbody(t, _):
        kA = 2 * t
        kB = kA + 1
        load_unpack(kB, iviB, ivjB)
        dB1 = pltpu.async_copy(h_h.at[ivjB], hjB, semB1)
        dB2 = pltpu.async_copy(h_h.at[iviB], hiB, semB2)
        pltpu.make_async_copy(h_h.at[ivjA], hjA, semA1).wait()
        pltpu.make_async_copy(h_h.at[iviA], hiA, semA2).wait()
        mul_scatter(kA, hjA, hiA, iviA, ivjA)
        kp = jnp.minimum(kA + 2, ECAP_W // CB - 1)
        load_unpack(kp, iviA, ivjA)
        dA1 = pltpu.async_copy(h_h.at[ivjA], hjA, semA1)
        dA2 = pltpu.async_copy(h_h.at[iviA], hiA, semA2)
        dB1.wait()
        dB2.wait()
        mul_scatter(kB, hjB, hiB, iviB, ivjB)
        return _

    lax.fori_loop(0, npairs, pairbody, 0)
    pltpu.make_async_copy(h_h.at[ivjA], hjA, semA1).wait()
    pltpu.make_async_copy(h_h.at[iviA], hiA, semA2).wait()
    plsc.subcore_barrier()
    pltpu.sync_copy(shared.at[pl.ds(sid * RPS, RPS)],
                    p2_h.at[cid, pl.ds(sid * RPS, RPS)])


_msg_pass = pl.kernel(
    _msg_body,
    out_type=[jax.ShapeDtypeStruct((NC, N, D), jnp.float32)],
    mesh=_mesh,
    scratch_types=[
        pltpu.VMEM((CB,), jnp.int32),
        pltpu.VMEM((CB,), jnp.int32),
        pltpu.VMEM((CB,), jnp.int32),
        pltpu.VMEM((CB,), jnp.int32),
        pltpu.VMEM((CB,), jnp.int32),
        pltpu.VMEM((CB, D), jnp.float32),
        pltpu.VMEM((CB, D), jnp.float32),
        pltpu.VMEM((CB, D), jnp.float32),
        pltpu.VMEM((CB, D), jnp.float32),
        pltpu.VMEM((CB, D), jnp.float32),
        pltpu.VMEM((L,), jnp.int32),
        pltpu.VMEM_SHARED((N, D), jnp.float32),
        pltpu.SemaphoreType.DMA,
        pltpu.SemaphoreType.DMA,
        pltpu.SemaphoreType.DMA,
        pltpu.SemaphoreType.DMA,
    ],
)


_BE = 2048


def _filt_tc_body(dq_ref, ij_ref, wf_ref, out_ref):
    d = jnp.sqrt(dq_ref[...] + 1e-12)
    mu = (CUTOFF / (N_RBF - 1)) * lax.broadcasted_iota(jnp.int32, (1, N_RBF), 1).astype(jnp.float32)
    rbf = jnp.exp(-GAMMA * (d - mu) ** 2)
    valid = (ij_ref[...] != 0).astype(jnp.float32)
    filt = jnp.dot(rbf, wf_ref[...], preferred_element_type=jnp.float32,
                   precision=lax.Precision.HIGHEST)
    out_ref[...] = filt * valid


def _filt_tc(dq2, ij2, W_filt):
    return pl.pallas_call(
        _filt_tc_body,
        grid=(E_CAP // _BE,),
        in_specs=[
            pl.BlockSpec((_BE, 1), lambda b: (b, 0)),
            pl.BlockSpec((_BE, 1), lambda b: (b, 0)),
            pl.BlockSpec((N_RBF, D), lambda b: (0, 0)),
        ],
        out_specs=pl.BlockSpec((_BE, D), lambda b: (b, 0)),
        out_shape=jax.ShapeDtypeStruct((E_CAP, D), jnp.float32),
    )(dq2, ij2, W_filt)


def _embed_tc_body(z_ref, emb_ref, wm_ref, wu_ref, h0_ref, wc_ref):
    zz = z_ref[...]
    onehot = (zz == lax.broadcasted_iota(jnp.int32, (1, NUM_SPECIES), 1)).astype(jnp.float32)
    h0_ref[...] = jnp.dot(onehot, emb_ref[...], preferred_element_type=jnp.float32,
                          precision=lax.Precision.HIGHEST)
    wc_ref[...] = jnp.dot(wm_ref[...], wu_ref[...], preferred_element_type=jnp.float32,
                          precision=lax.Precision.HIGHEST)


def _embed_tc(z2, emb, W_msg, W_upd):
    return pl.pallas_call(
        _embed_tc_body,
        out_shape=[
            jax.ShapeDtypeStruct((N, D), jnp.float32),
            jax.ShapeDtypeStruct((D, D), jnp.float32),
        ],
    )(z2, emb, W_msg, W_upd)


_BU = 512


def _update_tc_body(p0_ref, p1_ref, h_ref, wc_ref, out_ref):
    pre = p0_ref[...] + p1_ref[...]
    agg = jnp.dot(pre, wc_ref[...], preferred_element_type=jnp.float32,
                  precision=lax.Precision.HIGHEST)
    out_ref[...] = h_ref[...] + jax.nn.silu(agg)


def _update_tc(p0, p1, h, Wc):
    return pl.pallas_call(
        _update_tc_body,
        grid=(N // _BU,),
        in_specs=[
            pl.BlockSpec((_BU, D), lambda b: (b, 0)),
            pl.BlockSpec((_BU, D), lambda b: (b, 0)),
            pl.BlockSpec((_BU, D), lambda b: (b, 0)),
            pl.BlockSpec((D, D), lambda b: (0, 0)),
        ],
        out_specs=pl.BlockSpec((_BU, D), lambda b: (b, 0)),
        out_shape=jax.ShapeDtypeStruct((N, D), jnp.float32),
    )(p0, p1, h, Wc)


def _energy_tc_body(h_ref, wo_ref, out_ref):
    e = jnp.dot(jax.nn.silu(h_ref[...]), wo_ref[...], preferred_element_type=jnp.float32,
                precision=lax.Precision.HIGHEST)
    out_ref[...] = jnp.sum(e).reshape(1, 1, 1)


def _energy_tc(h, W_out):
    return pl.pallas_call(
        _energy_tc_body,
        grid=(N // _BU,),
        in_specs=[
            pl.BlockSpec((_BU, D), lambda b: (b, 0)),
            pl.BlockSpec((D, 1), lambda b: (0, 0)),
        ],
        out_specs=pl.BlockSpec((1, 1, 1), lambda b: (b, 0, 0)),
        out_shape=jax.ShapeDtypeStruct((N // _BU, 1, 1), jnp.float32),
    )(h, W_out)


def kernel(q, z, emb, W_filt, W_msg, W_upd, W_out):
    qx = q[:, 0]
    qy = q[:, 1]
    qz = q[:, 2]
    sx = jnp.broadcast_to(qx[:, None], (N, L))
    sy = jnp.broadcast_to(qy[:, None], (N, L))
    sz = jnp.broadcast_to(qz[:, None], (N, L))

    ij, dq, cnt = _pair_search(qx, qy, qz, sx, sy, sz)

    filt = _filt_tc(dq[:, None], ij[:, None], W_filt)
    h, Wc = _embed_tc(z[:, None].astype(jnp.int32), emb, W_msg, W_upd)

    for _ in range(2):
        (p2,) = _msg_pass(h, ij, filt, cnt)
        h = _update_tc(p2[0], p2[1], h, Wc)

    eparts = _energy_tc(h, W_out)
    return jnp.sum(eparts)


# speedup vs baseline: 1.6959x; 1.6959x over previous
import functools

import jax
import jax.numpy as jnp
from jax import lax
from jax.experimental import pallas as pl
from jax.experimental.pallas import tpu as pltpu
from jax.experimental.pallas import tpu_sc as plsc

N = 4096
D = 128
N_RBF = 64
NUM_SPECIES = 8
CELL = 1.0
CUTOFF = 0.12
CUT2 = CUTOFF * CUTOFF
GAMMA = 0.5 / ((CUTOFF / N_RBF) ** 2)

NC = 2
NS = 16
L = 16
NW = NC * NS
ROWS_PER_W = N // NW
ECAP_W = 4096
E_CAP = NW * ECAP_W
HALF = N // 2
RING_CHUNKS = HALF // L
SC_SLOTS = NS * ECAP_W
TRASH_SP = SC_SLOTS

CB = 128
RPS = N // NS

_mesh = plsc.VectorSubcoreMesh(core_axis_name="c", subcore_axis_name="s")


def _pair_body(qx_h, qy_h, qz_h, sx_h, sy_h, sz_h,
               ij_h, dq_h, cnt_h,
               qx_v, qy_v, qz_v, sx_v, sy_v, sz_v,
               stage, zb_v, zf_v, vv_ij, vv_d, cnt_v,
               sh_ij, sh_dq, sem_i, sem_d):
    cid = lax.axis_index("c").astype(jnp.int32)
    sid = lax.axis_index("s").astype(jnp.int32)
    wid = sid * NC + cid
    row0 = wid * ROWS_PER_W
    s0 = sid * ECAP_W

    pltpu.sync_copy(qx_h, qx_v.at[pl.ds(0, N)])
    pltpu.sync_copy(qy_h, qy_v.at[pl.ds(0, N)])
    pltpu.sync_copy(qz_h, qz_v.at[pl.ds(0, N)])
    pltpu.sync_copy(qx_h.at[pl.ds(0, HALF + L)], qx_v.at[pl.ds(N, HALF + L)])
    pltpu.sync_copy(qy_h.at[pl.ds(0, HALF + L)], qy_v.at[pl.ds(N, HALF + L)])
    pltpu.sync_copy(qz_h.at[pl.ds(0, HALF + L)], qz_v.at[pl.ds(N, HALF + L)])
    pltpu.sync_copy(sx_h.at[pl.ds(row0, ROWS_PER_W)], sx_v)
    pltpu.sync_copy(sy_h.at[pl.ds(row0, ROWS_PER_W)], sy_v)
    pltpu.sync_copy(sz_h.at[pl.ds(row0, ROWS_PER_W)], sz_v)

    stage[...] = jnp.zeros((160,), jnp.int32)
    zb_v[...] = jnp.zeros((512,), jnp.int32)
    zf_v[...] = jnp.zeros((512,), jnp.float32)

    def zfill(t, _):
        off = s0 + t * 512
        pltpu.sync_copy(zb_v, sh_ij.at[pl.ds(off, 512)])
        pltpu.sync_copy(zf_v, sh_dq.at[pl.ds(off, 512)])
        return _

    lax.fori_loop(0, ECAP_W // 512, zfill, 0)
    iota = lax.iota(jnp.int32, L)

    def ibody(k, cnt):
        i = row0 + k
        qix = sx_v[k, pl.ds(0, L)]
        qiy = sy_v[k, pl.ds(0, L)]
        qiz = sz_v[k, pl.ds(0, L)]
        ihi = jnp.full((L,), i * 4096, jnp.int32)
        anti_keep = i < HALF

        def dist(o):
            dx = qx_v[pl.ds(o, L)] - qix
            dy = qy_v[pl.ds(o, L)] - qiy
            dz = qz_v[pl.ds(o, L)] - qiz
            ax = 1.0 - jnp.abs(dx)
            ay = 1.0 - jnp.abs(dy)
            az = 1.0 - jnp.abs(dz)
            w = jnp.minimum(dx * dx, ax * ax)
            w = w + jnp.minimum(dy * dy, ay * ay)
            w = w + jnp.minimum(dz * dz, az * az)
            return w

        def emit(o, dsq, m, c_):
            mi = jnp.where(m, 1, 0)
            s = mi
            p = mi
            for sh in (1, 2, 4, 8):
                stage[pl.ds(80, L)] = s
                s = s + stage[pl.ds(80 + sh, L)]
                stage[pl.ds(48, L)] = p
                p = p + stage[pl.ds(48 - sh, L)]
            total = s[0]
            room = jnp.minimum(total, ECAP_W - c_)

            @pl.when(total > 0)
            def _():
                ok = m & (p <= jnp.full((L,), ECAP_W - c_, jnp.int32))
                idx = jnp.where(ok, jnp.full((L,), s0 + c_ - 1, jnp.int32) + p,
                                jnp.full((L,), TRASH_SP, jnp.int32))
                jv = (o + iota) & (N - 1)
                vv_ij[...] = ihi | jv
                vv_d[...] = dsq
                pltpu.async_copy(vv_ij, sh_ij.at[idx], sem_i, add=True).wait()
                pltpu.async_copy(vv_d, sh_dq.at[idx], sem_d, add=True).wait()

            return c_ + room

        def upair(u, c):
            oA = i + 1 + u * 2 * L
            oB = oA + L
            dsqA = dist(oA)
            dsqB = dist(oB)
            mA = (dsqA < CUT2) & (dsqA != 0.0)
            mB = (dsqB < CUT2) & (dsqB != 0.0)
            lim = jnp.where((u < RING_CHUNKS // 2 - 1) | anti_keep, 16, 15)
            mB = mB & (iota < jnp.full((L,), lim, jnp.int32))
            mor = jnp.where(mA | mB, 1, 0)
            s = mor
            for sh in (1, 2, 4, 8):
                stage[pl.ds(112, L)] = s
                s = s | stage[pl.ds(112 + sh, L)]
            anyhit = s[0]

            def scat2(c_):
                c2 = emit(oA, dsqA, mA, c_)
                return emit(oB, dsqB, mB, c2)

            return lax.cond(anyhit > 0, scat2, lambda c_: c_, c)

        return lax.fori_loop(0, RING_CHUNKS // 2, upair, cnt)

    cnt = lax.fori_loop(0, ROWS_PER_W, ibody, jnp.int32(0))
    cnt_v[...] = jnp.full((L,), cnt, jnp.int32)
    pltpu.sync_copy(cnt_v, cnt_h.at[wid])
    e0 = wid * ECAP_W
    pltpu.sync_copy(sh_ij.at[pl.ds(s0, ECAP_W)], ij_h.at[pl.ds(e0, ECAP_W)])
    pltpu.sync_copy(sh_dq.at[pl.ds(s0, ECAP_W)], dq_h.at[pl.ds(e0, ECAP_W)])


_pair_search = pl.kernel(
    _pair_body,
    out_type=[
        jax.ShapeDtypeStruct((E_CAP,), jnp.int32),
        jax.ShapeDtypeStruct((E_CAP,), jnp.float32),
        jax.ShapeDtypeStruct((NW, L), jnp.int32),
    ],
    mesh=_mesh,
    scratch_types=[
        pltpu.VMEM((N + HALF + L,), jnp.float32),
        pltpu.VMEM((N + HALF + L,), jnp.float32),
        pltpu.VMEM((N + HALF + L,), jnp.float32),
        pltpu.VMEM((ROWS_PER_W, L), jnp.float32),
        pltpu.VMEM((ROWS_PER_W, L), jnp.float32),
        pltpu.VMEM((ROWS_PER_W, L), jnp.float32),
        pltpu.VMEM((160,), jnp.int32),
        pltpu.VMEM((512,), jnp.int32),
        pltpu.VMEM((512,), jnp.float32),
        pltpu.VMEM((L,), jnp.int32),
        pltpu.VMEM((L,), jnp.float32),
        pltpu.VMEM((L,), jnp.int32),
        pltpu.VMEM_SHARED((SC_SLOTS + L,), jnp.int32),
        pltpu.VMEM_SHARED((SC_SLOTS + L,), jnp.float32),
        pltpu.SemaphoreType.DMA,
        pltpu.SemaphoreType.DMA,
    ],
)


def _msg_body(h_h, ij_h, filt_h, cnt_h, p2_h,
              ijv, iviA, ivjA, hjA, hiA, ft, cntv, shared,
              semA1, semA2):
    cid = lax.axis_index("c").astype(jnp.int32)
    sid = lax.axis_index("s").astype(jnp.int32)
    wid = sid * NC + cid
    e0 = wid * ECAP_W

    hjA[...] = jnp.zeros((CB, D), jnp.float32)
    pltpu.sync_copy(hjA, shared.at[pl.ds(sid * RPS, CB)])
    pltpu.sync_copy(hjA, shared.at[pl.ds(sid * RPS + CB, CB)])
    plsc.subcore_barrier()

    pltpu.sync_copy(cnt_h.at[wid], cntv)
    c = jnp.minimum(cntv[pl.ds(0, L)][0], ECAP_W)
    nch = (c + CB - 1) // CB

    def load_unpack(k, ivi, ivj):
        pltpu.sync_copy(ij_h.at[pl.ds(e0 + k * CB, CB)], ijv)
        for g in range(CB // L):
            v = ijv[pl.ds(g * L, L)]
            ivi[pl.ds(g * L, L)] = lax.shift_right_logical(v, 12)
            ivj[pl.ds(g * L, L)] = v & (N - 1)

    def mul_scatter(k, hj, hi, ivi, ivj):
        def rowbody(r, __):
            for col in range(0, D, L):
                hj[r, pl.ds(col, L)] = hj[r, pl.ds(col, L)] * ft[r, pl.ds(col, L)]
                hi[r, pl.ds(col, L)] = hi[r, pl.ds(col, L)] * ft[r, pl.ds(col, L)]
            return __

        lax.fori_loop(0, CB, rowbody, 0)
        pltpu.sync_copy(hj, shared.at[ivi], add=True)
        pltpu.sync_copy(hi, shared.at[ivj], add=True)

    def chunk(k, _):
        load_unpack(k, iviA, ivjA)
        dA1 = pltpu.async_copy(h_h.at[ivjA], hjA, semA1)
        dA2 = pltpu.async_copy(h_h.at[iviA], hiA, semA2)
        pltpu.sync_copy(filt_h.at[pl.ds(e0 + k * CB, CB)], ft)
        dA1.wait()
        dA2.wait()
        mul_scatter(k, hjA, hiA, iviA, ivjA)
        return _

    lax.fori_loop(0, nch, chunk, 0)
    plsc.subcore_barrier()
    pltpu.sync_copy(shared.at[pl.ds(sid * RPS, RPS)],
                    p2_h.at[cid, pl.ds(sid * RPS, RPS)])


_msg_pass = pl.kernel(
    _msg_body,
    out_type=[jax.ShapeDtypeStruct((NC, N, D), jnp.float32)],
    mesh=_mesh,
    scratch_types=[
        pltpu.VMEM((CB,), jnp.int32),
        pltpu.VMEM((CB,), jnp.int32),
        pltpu.VMEM((CB,), jnp.int32),
        pltpu.VMEM((CB, D), jnp.float32),
        pltpu.VMEM((CB, D), jnp.float32),
        pltpu.VMEM((CB, D), jnp.float32),
        pltpu.VMEM((L,), jnp.int32),
        pltpu.VMEM_SHARED((N, D), jnp.float32),
        pltpu.SemaphoreType.DMA,
        pltpu.SemaphoreType.DMA,
    ],
)


_BE = 2048


def _filt_tc_body(dq_ref, ij_ref, wf_ref, out_ref):
    d = jnp.sqrt(dq_ref[...] + 1e-12)
    mu = (CUTOFF / (N_RBF - 1)) * lax.broadcasted_iota(jnp.int32, (1, N_RBF), 1).astype(jnp.float32)
    rbf = jnp.exp(-GAMMA * (d - mu) ** 2)
    valid = (ij_ref[...] != 0).astype(jnp.float32)
    filt = jnp.dot(rbf, wf_ref[...], preferred_element_type=jnp.float32,
                   precision=lax.Precision.HIGHEST)
    out_ref[...] = filt * valid


def _filt_tc(dq2, ij2, W_filt):
    return pl.pallas_call(
        _filt_tc_body,
        grid=(E_CAP // _BE,),
        in_specs=[
            pl.BlockSpec((_BE, 1), lambda b: (b, 0)),
            pl.BlockSpec((_BE, 1), lambda b: (b, 0)),
            pl.BlockSpec((N_RBF, D), lambda b: (0, 0)),
        ],
        out_specs=pl.BlockSpec((_BE, D), lambda b: (b, 0)),
        out_shape=jax.ShapeDtypeStruct((E_CAP, D), jnp.float32),
    )(dq2, ij2, W_filt)


def _embed_tc_body(z_ref, emb_ref, wm_ref, wu_ref, h0_ref, wc_ref):
    zz = z_ref[...]
    onehot = (zz == lax.broadcasted_iota(jnp.int32, (1, NUM_SPECIES), 1)).astype(jnp.float32)
    h0_ref[...] = jnp.dot(onehot, emb_ref[...], preferred_element_type=jnp.float32,
                          precision=lax.Precision.HIGHEST)
    wc_ref[...] = jnp.dot(wm_ref[...], wu_ref[...], preferred_element_type=jnp.float32,
                          precision=lax.Precision.HIGHEST)


def _embed_tc(z2, emb, W_msg, W_upd):
    return pl.pallas_call(
        _embed_tc_body,
        out_shape=[
            jax.ShapeDtypeStruct((N, D), jnp.float32),
            jax.ShapeDtypeStruct((D, D), jnp.float32),
        ],
    )(z2, emb, W_msg, W_upd)


_BU = 512


def _update_tc_body(p0_ref, p1_ref, h_ref, wc_ref, out_ref):
    pre = p0_ref[...] + p1_ref[...]
    agg = jnp.dot(pre, wc_ref[...], preferred_element_type=jnp.float32,
                  precision=lax.Precision.HIGHEST)
    out_ref[...] = h_ref[...] + jax.nn.silu(agg)


def _update_tc(p0, p1, h, Wc):
    return pl.pallas_call(
        _update_tc_body,
        grid=(N // _BU,),
        in_specs=[
            pl.BlockSpec((_BU, D), lambda b: (b, 0)),
            pl.BlockSpec((_BU, D), lambda b: (b, 0)),
            pl.BlockSpec((_BU, D), lambda b: (b, 0)),
            pl.BlockSpec((D, D), lambda b: (0, 0)),
        ],
        out_specs=pl.BlockSpec((_BU, D), lambda b: (b, 0)),
        out_shape=jax.ShapeDtypeStruct((N, D), jnp.float32),
    )(p0, p1, h, Wc)


def _energy_tc_body(h_ref, wo_ref, out_ref):
    e = jnp.dot(jax.nn.silu(h_ref[...]), wo_ref[...], preferred_element_type=jnp.float32,
                precision=lax.Precision.HIGHEST)
    out_ref[...] = jnp.sum(e).reshape(1, 1, 1)


def _energy_tc(h, W_out):
    return pl.pallas_call(
        _energy_tc_body,
        grid=(N // _BU,),
        in_specs=[
            pl.BlockSpec((_BU, D), lambda b: (b, 0)),
            pl.BlockSpec((D, 1), lambda b: (0, 0)),
        ],
        out_specs=pl.BlockSpec((1, 1, 1), lambda b: (b, 0, 0)),
        out_shape=jax.ShapeDtypeStruct((N // _BU, 1, 1), jnp.float32),
    )(h, W_out)


def kernel(q, z, emb, W_filt, W_msg, W_upd, W_out):
    qx = q[:, 0]
    qy = q[:, 1]
    qz = q[:, 2]
    sx = jnp.broadcast_to(qx[:, None], (N, L))
    sy = jnp.broadcast_to(qy[:, None], (N, L))
    sz = jnp.broadcast_to(qz[:, None], (N, L))

    ij, dq, cnt = _pair_search(qx, qy, qz, sx, sy, sz)

    filt = _filt_tc(dq[:, None], ij[:, None], W_filt)
    h, Wc = _embed_tc(z[:, None].astype(jnp.int32), emb, W_msg, W_upd)

    for _ in range(2):
        (p2,) = _msg_pass(h, ij, filt, cnt)
        h = _update_tc(p2[0], p2[1], h, Wc)

    eparts = _energy_tc(h, W_out)
    return jnp.sum(eparts)
